# Initial kernel scaffold; baseline (speedup 1.0000x reference)
import jax
import jax.numpy as jnp
from jax.experimental import pallas as pl


def kernel(objectness, pred_bbox_deltas, anchors):
    raise NotImplementedError("write your pallas kernel here")


# speedup vs baseline: 16.5009x; 16.5009x over previous
import numpy as np
import jax
import jax.numpy as jnp
from jax import lax
from jax.experimental import pallas as pl
from jax.experimental.pallas import tpu as pltpu

_K = 1000
_NMS_THRESH = np.float32(0.7)
_MIN_SIZE = np.float32(0.001)
_NEG = np.float32(-1e9)
_NEG8 = np.float32(-1e8)
_N = 30000
_NPAD = 30720
_NR = 240
_NL = 128
_S = 1024
_SA = 1152
_F = 16

_ATAN_C = [0.9999999841273948, -0.3333319473578165, 0.19996625981596905,
           -0.14248418056055476, 0.10882235117231387, -0.08222721956096983,
           0.055144646975996324, -0.028581811743819928, 0.00960651932071977,
           -0.0015164396613546747]

_HI = lax.Precision.HIGHEST


def _atan2(y, x):
    ax, ay = jnp.abs(x), jnp.abs(y)
    hi = jnp.maximum(ax, ay)
    lo = jnp.minimum(ax, ay)
    t = lo / jnp.maximum(hi, np.float32(1e-30))
    t2 = t * t
    acc = jnp.full_like(t, np.float32(_ATAN_C[-1]))
    for k in range(len(_ATAN_C) - 2, -1, -1):
        acc = acc * t2 + np.float32(_ATAN_C[k])
    a = acc * t
    a = jnp.where(ay > ax, np.float32(np.pi / 2) - a, a)
    a = jnp.where(x < 0, np.float32(np.pi) - a, a)
    return jnp.where(y < 0, -a, a)


def _dot(a, b, dims):
    return lax.dot_general(a, b, (dims, ((), ())), precision=_HI,
                           preferred_element_type=jnp.float32)


def _body(obj_ref, vals_ref, out_ref, p_ref, sel_ref, acc_ref, rnk_ref,
          r2_ref, sup_ref):
    f32 = jnp.float32
    i32 = jnp.int32

    obj2 = obj_ref[0]
    bits = lax.bitcast_convert_type(obj2, i32)
    minint = i32(-2**31)
    key = jnp.where(bits >= 0, bits, minint - bits)

    kf = np.float32(_K)

    def _tbody(i, t):
        cand = t + (i32(1) << (i32(31) - i))
        cnt = jnp.sum((key >= cand).astype(f32))
        return jnp.where(cnt >= kf, cand, t)
    t = lax.fori_loop(0, 32, _tbody, minint)

    m = jnp.sum((key > t).astype(f32))
    r = kf - m

    idx2 = (lax.broadcasted_iota(i32, (_NR, _NL), 0) * _NL
            + lax.broadcasted_iota(i32, (_NR, _NL), 1))
    ties = key == t

    def _cbody(i, c):
        cand = c + (i32(1) << (i32(15) - i))
        cnt = jnp.sum((ties & (idx2 < cand)).astype(f32))
        return jnp.where(cnt <= r, cand, c)
    c = lax.fori_loop(0, 16, _cbody, i32(0))

    sel = (key > t) | (ties & (idx2 < c))
    self_f = sel.astype(f32)
    sel_ref[...] = self_f

    li = lax.broadcasted_iota(i32, (_NL, _NL), 0)
    lj = lax.broadcasted_iota(i32, (_NL, _NL), 1)
    u_excl = (li < lj).astype(f32)
    ri = lax.broadcasted_iota(i32, (_NR, _NR), 0)
    rj = lax.broadcasted_iota(i32, (_NR, _NR), 1)
    t240 = (rj < ri).astype(f32)
    cs = _dot(self_f, u_excl, ((1,), (0,)))
    rowsum = jnp.sum(self_f, axis=1, keepdims=True)
    offs = _dot(t240, rowsum, ((1,), (0,)))
    p_ref[...] = cs + offs

    acc_ref[...] = jnp.zeros((_SA, _F), f32)
    w_iota = lax.broadcasted_iota(i32, (136, _NL), 0)

    def _scat(rr, _):
        offr = p_ref[rr, 0]
        off8 = lax.bitwise_and(offr.astype(i32), i32(-8))
        dloc = p_ref[pl.ds(rr, 1), :].astype(i32) - off8
        selr = sel_ref[pl.ds(rr, 1), :]
        oh = jnp.where((w_iota == dloc) & (selr > 0.5), f32(1), f32(0))
        vr = vals_ref[0, :, pl.ds(rr * _NL, _NL)]
        contrib = _dot(oh, vr, ((1,), (1,)))
        cur = acc_ref[pl.ds(off8, 136), :]
        acc_ref[pl.ds(off8, 136), :] = cur + contrib
        return 0
    lax.fori_loop(0, _NR, _scat, 0)

    compact = jnp.transpose(acc_ref[0:_S, :])

    lane_s = lax.broadcasted_iota(i32, (1, _S), 1)
    slot_real = lane_s < _K

    obj_r = compact[0:1, :]
    dx, dy, dw, dh = (compact[1:2], compact[2:3], compact[3:4], compact[4:5])
    dcos, dsin = compact[5:6], compact[6:7]
    xa, ya, wa, ha, aa = (compact[7:8], compact[8:9], compact[9:10],
                          compact[10:11], compact[11:12])
    px = dx * wa + xa
    py = dy * ha + ya
    pw = jnp.exp(jnp.minimum(dw, f32(4.0))) * wa
    ph = jnp.exp(jnp.minimum(dh, f32(4.0))) * ha
    pa = aa + jnp.degrees(jnp.arctan2(dsin, dcos))

    score = f32(1.0) / (f32(1.0) + jnp.exp(-obj_r))
    valid = (pw >= _MIN_SIZE) & (ph >= _MIN_SIZE) & (score >= f32(0.0))
    valid &= slot_real
    score_m = jnp.where(valid, score, _NEG)

    rad = pa * np.float32(np.pi / 180.0)
    cosr = jnp.abs(jnp.cos(rad))
    sinr = jnp.abs(jnp.sin(rad))
    ex = (pw * cosr + ph * sinr) * f32(0.5)
    ey = (pw * sinr + ph * cosr) * f32(0.5)
    bx1, by1, bx2, by2 = px - ex, py - ey, px + ex, py + ey

    obj_m = jnp.where(slot_real, obj_r, -jnp.inf)
    valid_f = valid.astype(f32)

    rnk_ref[0:1, :] = obj_m
    rnk_ref[1:2, :] = valid_f
    obj_c = jnp.transpose(obj_m)
    valid_c = jnp.transpose(valid_f)
    sub_i = lax.broadcasted_iota(i32, (_S, _NL), 0)

    def _rank(jh, _):
        j0 = jh * _NL
        objr = rnk_ref[0:1, pl.ds(j0, _NL)]
        lane_g = j0 + lax.broadcasted_iota(i32, (_S, _NL), 1)
        before = (obj_c > objr) | ((obj_c == objr) & (sub_i < lane_g))
        bf = before.astype(f32)
        rnk_ref[2:3, pl.ds(j0, _NL)] = jnp.sum(bf, axis=0, keepdims=True)
        rnk_ref[3:4, pl.ds(j0, _NL)] = jnp.sum(bf * valid_c, axis=0,
                                               keepdims=True)
        return 0
    lax.fori_loop(0, _S // _NL, _rank, 0)

    rank1 = rnk_ref[2:3, :]
    cnt_v = rnk_ref[3:4, :]
    n_v = jnp.sum(valid_f)
    pos2 = jnp.where(valid, cnt_v, n_v + rank1 - cnt_v)

    v2 = jnp.concatenate(
        [px, py, pw, ph, pa, score_m, bx1, by1, bx2, by2,
         jnp.zeros((6, _S), f32)], axis=0)
    pos2_c = jnp.transpose(pos2).astype(i32)
    lane_i = lax.broadcasted_iota(i32, (_S, _NL), 1)

    def _scat2(jh, _):
        j0 = jh * _NL
        w2 = jnp.where(pos2_c == lane_i + j0, f32(1), f32(0))
        r2_ref[:, pl.ds(j0, _NL)] = _dot(v2, w2, ((1,), (0,)))
        return 0
    lax.fori_loop(0, _S // _NL, _scat2, 0)

    r2 = r2_ref[...]
    area_r = (r2[8:9, :] - r2[6:7, :]) * (r2[9:10, :] - r2[7:8, :])
    r2_ref[10:11, :] = area_r

    x1c = jnp.transpose(r2[6:7, :])
    y1c = jnp.transpose(r2[7:8, :])
    x2c = jnp.transpose(r2[8:9, :])
    y2c = jnp.transpose(r2[9:10, :])
    area_c = (x2c - x1c) * (y2c - y1c)

    def _supb(jh, _):
        j0 = jh * _NL
        rows = r2_ref[6:11, pl.ds(j0, _NL)]
        ix1 = jnp.maximum(x1c, rows[0:1, :])
        iy1 = jnp.maximum(y1c, rows[1:2, :])
        ix2 = jnp.minimum(x2c, rows[2:3, :])
        iy2 = jnp.minimum(y2c, rows[3:4, :])
        inter = (jnp.maximum(ix2 - ix1, f32(0.0))
                 * jnp.maximum(iy2 - iy1, f32(0.0)))
        union = area_c + rows[4:5, :] - inter
        iou = inter / jnp.maximum(union, np.float32(1e-9))
        jg = j0 + lax.broadcasted_iota(i32, (_S, _NL), 1)
        supc = (iou > _NMS_THRESH) & (jg > lax.broadcasted_iota(
            i32, (_S, _NL), 0))
        sup_ref[:, pl.ds(j0, _NL)] = supc.astype(f32)
        return 0
    lax.fori_loop(0, _S // _NL, _supb, 0)

    def _nms(i, keep):
        row = sup_ref[pl.ds(i, 1), :]
        ohi = (lane_s == i).astype(f32)
        keep_i = jnp.sum(keep * ohi)
        return keep * (f32(1.0) - row * keep_i)
    keep_r = lax.fori_loop(0, _K, _nms, jnp.ones((1, _S), f32))

    score2 = r2[5:6, :]
    kept_r = keep_r * (score2 > _NEG8).astype(f32)

    si = lax.broadcasted_iota(i32, (_S, _S), 0)
    sj = lax.broadcasted_iota(i32, (_S, _S), 1)
    u_s = (si < sj).astype(f32)
    cnt_k = _dot(kept_r, u_s, ((1,), (0,)))
    n_k = jnp.sum(kept_r)
    cnt_u = lane_s.astype(f32) - cnt_k
    pos3 = jnp.where(kept_r > 0.5, cnt_k, n_k + cnt_u)
    pos3_c = jnp.transpose(pos3).astype(i32)

    masked = jnp.where(kept_r > 0.5, score2, _NEG)
    v3 = jnp.concatenate([r2[0:5, :], masked, jnp.zeros((2, _S), f32)],
                         axis=0)

    def _scat3(jh, _):
        j0 = jh * _NL
        w3 = jnp.where(pos3_c == lane_i + j0, f32(1), f32(0))
        out_ref[0, :, pl.ds(j0, _NL)] = _dot(v3, w3, ((1,), (0,)))
        return 0
    lax.fori_loop(0, _S // _NL, _scat3, 0)


def kernel(objectness, pred_bbox_deltas, anchors):
    B, A, H, W = objectness.shape
    N = A * H * W
    f32 = jnp.float32

    obj = jnp.transpose(objectness, (0, 2, 3, 1)).reshape(B, N)
    obj = jnp.pad(obj, ((0, 0), (0, _NPAD - N)),
                  constant_values=-np.inf).reshape(B, _NR, _NL)

    deltas = jnp.transpose(
        pred_bbox_deltas.reshape(B, A, 6, H, W), (0, 2, 3, 4, 1)
    ).reshape(B, 6, N)
    anch = jnp.transpose(anchors, (0, 2, 1))
    objrow = jnp.transpose(objectness, (0, 2, 3, 1)).reshape(B, 1, N)
    vals = jnp.concatenate(
        [objrow, deltas, anch, jnp.zeros((B, 4, N), f32)], axis=1)
    vals = jnp.pad(vals, ((0, 0), (0, 0), (0, _NPAD - N)))

    out = pl.pallas_call(
        _body,
        grid=(B,),
        in_specs=[
            pl.BlockSpec((1, _NR, _NL), lambda b: (b, 0, 0)),
            pl.BlockSpec((1, _F, _NPAD), lambda b: (b, 0, 0)),
        ],
        out_specs=pl.BlockSpec((1, 8, _S), lambda b: (b, 0, 0)),
        out_shape=jax.ShapeDtypeStruct((B, 8, _S), f32),
        scratch_shapes=[
            pltpu.VMEM((_NR, _NL), f32),
            pltpu.VMEM((_NR, _NL), f32),
            pltpu.VMEM((_SA, _F), f32),
            pltpu.VMEM((4, _S), f32),
            pltpu.VMEM((_F, _S), f32),
            pltpu.VMEM((_S, _S), f32),
        ],
    )(obj, vals)

    boxes = jnp.transpose(out[:, 0:5, 0:_K], (0, 2, 1))
    scores = out[:, 5, 0:_K]
    return boxes, scores
